# Initial kernel scaffold; baseline (speedup 1.0000x reference)
import jax
import jax.numpy as jnp
from jax.experimental import pallas as pl


def kernel(x, gate_w, expert_w1, expert_w2, expert_b1, expert_b2):
    raise NotImplementedError("write your pallas kernel here")


# speedup vs baseline: 5.1930x; 5.1930x over previous
import functools
import math

import jax
import jax.numpy as jnp
from jax import lax
from jax.experimental import pallas as pl
from jax.experimental.pallas import tpu as pltpu

TOK = 2048
DIM = 768
NE = 8
FFD = 1536
K = 2

RT = 256
GT = 256
NPAIR = K * TOK
NUM_TILES = (NPAIR + NE * (GT - 1) + GT - 1) // GT
PADDED = NUM_TILES * GT

_SQRT2 = math.sqrt(2.0)


def _router_body(x_ref, gw_ref, idx_ref, wts_ref, stats_ref):
    i = pl.program_id(0)
    x = x_ref[...]
    logits = jnp.dot(x, gw_ref[...], preferred_element_type=jnp.float32)

    m1 = jnp.max(logits, axis=1)
    i1 = jnp.argmax(logits, axis=1)
    col = lax.broadcasted_iota(jnp.int32, logits.shape, 1)
    masked = jnp.where(col == i1[:, None], -jnp.inf, logits)
    m2 = jnp.max(masked, axis=1)
    i2 = jnp.argmax(masked, axis=1)

    z = jnp.exp(m2 - m1)
    w1 = 1.0 / (1.0 + z)
    w2 = z * w1

    idx_ref[...] = jnp.stack([i1, i2], axis=1).astype(jnp.int32)
    wts_ref[...] = jnp.stack([w1, w2], axis=1)

    probs = jax.nn.softmax(logits, axis=1)
    onehot = (col == i1[:, None]).astype(jnp.float32)
    cnt = jnp.sum(onehot, axis=0)
    psum = jnp.sum(probs, axis=0)
    sq = jnp.sum(logits * logits)

    row = lax.broadcasted_iota(jnp.int32, (8, NE), 0)
    upd = jnp.where(row == 0, cnt[None, :],
                    jnp.where(row == 1, psum[None, :],
                              jnp.where(row == 2, sq, 0.0)))

    @pl.when(i == 0)
    def _():
        stats_ref[...] = jnp.zeros_like(stats_ref)

    stats_ref[...] += upd


def _ffn_body(te_ref, x_ref, w1_ref, w2_ref, b1_ref, b2_ref, o_ref):
    h = jnp.dot(x_ref[...], w1_ref[0], preferred_element_type=jnp.float32)
    h = h + b1_ref[0]
    h = 0.5 * h * (1.0 + lax.erf(h / _SQRT2))
    o = jnp.dot(h, w2_ref[0], preferred_element_type=jnp.float32)
    o_ref[...] = o + b2_ref[0]


def kernel(x, gate_w, expert_w1, expert_w2, expert_b1, expert_b2):
    xf = x.reshape(TOK, DIM)

    idx, wts, stats = pl.pallas_call(
        _router_body,
        grid=(TOK // RT,),
        in_specs=[
            pl.BlockSpec((RT, DIM), lambda i: (i, 0)),
            pl.BlockSpec((DIM, NE), lambda i: (0, 0)),
        ],
        out_specs=[
            pl.BlockSpec((RT, K), lambda i: (i, 0)),
            pl.BlockSpec((RT, K), lambda i: (i, 0)),
            pl.BlockSpec((8, NE), lambda i: (0, 0)),
        ],
        out_shape=[
            jax.ShapeDtypeStruct((TOK, K), jnp.int32),
            jax.ShapeDtypeStruct((TOK, K), jnp.float32),
            jax.ShapeDtypeStruct((8, NE), jnp.float32),
        ],
    )(xf, gate_w)

    cnt = stats[0]
    psum = stats[1]
    sq = stats[2, 0]
    aux_loss = NE * jnp.sum(cnt * psum) / (TOK * TOK)
    z_loss = sq / (TOK * NE) * 0.001
    total_aux = aux_loss + z_loss

    e_flat = idx.reshape(NPAIR)
    onehot = (e_flat[:, None] == jnp.arange(NE)[None, :]).astype(jnp.int32)
    counts = jnp.sum(onehot, axis=0)
    rank = jnp.cumsum(onehot, axis=0) - onehot
    rank_p = jnp.sum(rank * onehot, axis=1)
    pcounts = ((counts + GT - 1) // GT) * GT
    pstart = jnp.concatenate([jnp.zeros((1,), jnp.int32),
                              jnp.cumsum(pcounts)[:-1].astype(jnp.int32)])
    pos = pstart[e_flat] + rank_p

    tok_of_pair = jnp.arange(NPAIR, dtype=jnp.int32) // K
    gather_idx = jnp.zeros((PADDED,), jnp.int32).at[pos].set(tok_of_pair)

    pend_tiles = (jnp.cumsum(pcounts) // GT).astype(jnp.int32)
    g = jnp.arange(NUM_TILES, dtype=jnp.int32)
    tile_expert = jnp.sum(
        (pend_tiles[None, :] <= g[:, None]).astype(jnp.int32), axis=1)
    tile_expert = jnp.minimum(tile_expert, NE - 1)

    x_sorted = jnp.take(xf, gather_idx, axis=0)

    res = pl.pallas_call(
        _ffn_body,
        grid_spec=pltpu.PrefetchScalarGridSpec(
            num_scalar_prefetch=1,
            grid=(NUM_TILES,),
            in_specs=[
                pl.BlockSpec((GT, DIM), lambda i, te: (i, 0)),
                pl.BlockSpec((1, DIM, FFD), lambda i, te: (te[i], 0, 0)),
                pl.BlockSpec((1, FFD, DIM), lambda i, te: (te[i], 0, 0)),
                pl.BlockSpec((1, 1, FFD), lambda i, te: (te[i], 0, 0)),
                pl.BlockSpec((1, 1, DIM), lambda i, te: (te[i], 0, 0)),
            ],
            out_specs=pl.BlockSpec((GT, DIM), lambda i, te: (i, 0)),
        ),
        out_shape=jax.ShapeDtypeStruct((PADDED, DIM), jnp.float32),
    )(tile_expert, x_sorted, expert_w1, expert_w2,
      expert_b1.reshape(NE, 1, FFD), expert_b2.reshape(NE, 1, DIM))

    pos0 = pos[0::K]
    pos1 = pos[1::K]
    out = wts[:, 0:1] * jnp.take(res, pos0, axis=0) \
        + wts[:, 1:2] * jnp.take(res, pos1, axis=0)
    return (out.reshape(x.shape), total_aux)
